# Initial kernel scaffold; baseline (speedup 1.0000x reference)
import jax
import jax.numpy as jnp
from jax.experimental import pallas as pl


def kernel(Z, r, idx_i, idx_j, c6ab_flat, rcov, r2r4):
    raise NotImplementedError("write your pallas kernel here")


# speedup vs baseline: 1.1871x; 1.1871x over previous
import functools

import jax
import jax.numpy as jnp
from jax import lax
from jax.experimental import pallas as pl
from jax.experimental.pallas import tpu as pltpu

D3_A1 = 0.3385
D3_A2 = 2.883
D3_K1 = 16.0
D3_K3 = -4.0
D3_S6 = 1.0
D3_S8 = 0.9171
D3_MAXC2 = 25
EPS = 1e-10

LANES = 128
VMEM_LIMIT = 64 * 1024 * 1024


def _round_up(x, m):
    return ((x + m - 1) // m) * m


def _segsum_matmul(vals, idx, out_ref, nh):
    pt = idx.shape[-1]
    hi = idx >> 7
    lo = idx & 127
    ioh = lax.broadcasted_iota(jnp.int32, (nh, pt), 0)
    iol = lax.broadcasted_iota(jnp.int32, (LANES, pt), 0)
    a = jnp.where(hi == ioh, 1.0, 0.0)
    b = jnp.where(lo == iol, vals, 0.0)
    out_ref[...] += lax.dot_general(
        a, b, dimension_numbers=(((1,), (1,)), ((), ())),
        preferred_element_type=jnp.float32)


def _nc_kernel(r_ref, rco_ref, idx_ref, out_ref, *, nh):
    @pl.when(pl.program_id(1) == 0)
    def _init():
        out_ref[...] = jnp.zeros_like(out_ref)

    rr = rco_ref[...] / r_ref[...]
    damp = 1.0 / (1.0 + jnp.exp(-D3_K1 * (rr - 1.0)))
    _segsum_matmul(damp, idx_ref[...], out_ref, nh)


def _pair_energy_kernel(r_ref, r2r4_ref, nci_ref, ncj_ref, tab_ref, e_ref):
    r = r_ref[...]
    nci = nci_ref[...]
    ncj = ncj_ref[...]

    cn0 = tab_ref[0]
    cn1 = tab_ref[1]
    cn2 = tab_ref[2]

    rdist = (cn1 - nci[None]) ** 2 + (cn2 - ncj[None]) ** 2
    logits = D3_K3 * rdist
    m = jnp.max(logits, axis=0)
    w = jnp.exp(logits - m[None])
    wsum = jnp.sum(w, axis=0)
    c6 = jnp.sum(w * cn0, axis=0) / wsum

    c8 = 3.0 * c6 * r2r4_ref[...]
    r2 = r * r
    r6 = r2 * r2 * r2
    r8 = r6 * r2
    tmp = D3_A1 * jnp.sqrt(c8 / (c6 + EPS) + EPS) + D3_A2
    tmp2 = tmp * tmp
    tmp6 = tmp2 * tmp2 * tmp2
    tmp8 = tmp6 * tmp2
    e6 = -0.5 * D3_S6 * c6 / (r6 + tmp6)
    e8 = -0.5 * D3_S8 * c8 / (r8 + tmp8)
    e_ref[...] = e6 + e8


def _esum_kernel(e_ref, idx_ref, out_ref, *, nh):
    @pl.when(pl.program_id(1) == 0)
    def _init():
        out_ref[...] = jnp.zeros_like(out_ref)

    _segsum_matmul(e_ref[...], idx_ref[...], out_ref, nh)


def kernel(Z, r, idx_i, idx_j, c6ab_flat, rcov, r2r4):
    N = Z.shape[0]
    P = r.shape[0]
    MAXZ = rcov.shape[0]

    N_pad = _round_up(N, LANES)
    NH = N_pad // LANES

    PT = min(4096, _round_up(P, LANES))
    P_pad = _round_up(P, 2 * PT)
    n_half = P_pad // (2 * PT)
    PT2 = min(8192, P_pad)
    while P_pad % PT2:
        PT2 //= 2
    S_all = P_pad // LANES
    S_T2 = PT2 // LANES
    n2 = P_pad // PT2
    pad_p = P_pad - P

    idx_i = idx_i.astype(jnp.int32)
    idx_j = idx_j.astype(jnp.int32)
    idx_i_p = jnp.pad(idx_i, (0, pad_p), constant_values=N_pad)
    idx_j_p = jnp.pad(idx_j, (0, pad_p), constant_values=N_pad)
    r_p = jnp.pad(r.astype(jnp.float32), (0, pad_p), constant_values=1.0)

    gi = jnp.clip(idx_i_p, 0, N - 1)
    gj = jnp.clip(idx_j_p, 0, N - 1)
    Zi = Z[gi]
    Zj = Z[gj]
    rco = (rcov[Zi] + rcov[Zj]).astype(jnp.float32)
    r2r4_ij = (r2r4[Zi] * r2r4[Zj]).astype(jnp.float32)
    tab = jnp.take(c6ab_flat, Zi * MAXZ + Zj, axis=1)
    tab_k = tab.reshape(3, D3_MAXC2, S_all, LANES)

    r_row = r_p.reshape(1, P_pad)
    rco_row = rco.reshape(1, P_pad)
    idx_row = idx_i_p.reshape(1, P_pad)

    seg_pair_spec = pl.BlockSpec((1, PT), lambda c, p: (0, c * n_half + p))
    seg_out_spec = pl.BlockSpec((None, NH, LANES), lambda c, p: (c, 0, 0))
    seg_params = pltpu.CompilerParams(
        dimension_semantics=("parallel", "arbitrary"),
        vmem_limit_bytes=VMEM_LIMIT)

    nc_parts = pl.pallas_call(
        functools.partial(_nc_kernel, nh=NH),
        out_shape=jax.ShapeDtypeStruct((2, NH, LANES), jnp.float32),
        grid=(2, n_half),
        in_specs=[seg_pair_spec, seg_pair_spec, seg_pair_spec],
        out_specs=seg_out_spec,
        compiler_params=seg_params,
    )(r_row, rco_row, idx_row)
    nc_flat = jnp.sum(nc_parts, axis=0).reshape(N_pad)

    nci = nc_flat[jnp.minimum(idx_i_p, N_pad - 1)]
    ncj = nc_flat[jnp.minimum(idx_j_p, N_pad - 1)]

    pair_spec = pl.BlockSpec((S_T2, LANES), lambda p: (p, 0))
    table_spec = pl.BlockSpec((3, D3_MAXC2, S_T2, LANES), lambda p: (0, 0, p, 0))
    e_2d = pl.pallas_call(
        _pair_energy_kernel,
        out_shape=jax.ShapeDtypeStruct((S_all, LANES), jnp.float32),
        grid=(n2,),
        in_specs=[pair_spec, pair_spec, pair_spec, pair_spec, table_spec],
        out_specs=pl.BlockSpec((S_T2, LANES), lambda p: (p, 0)),
        compiler_params=pltpu.CompilerParams(
            dimension_semantics=("parallel",),
            vmem_limit_bytes=VMEM_LIMIT),
    )(r_p.reshape(S_all, LANES), r2r4_ij.reshape(S_all, LANES),
      nci.reshape(S_all, LANES), ncj.reshape(S_all, LANES), tab_k)

    e_parts = pl.pallas_call(
        functools.partial(_esum_kernel, nh=NH),
        out_shape=jax.ShapeDtypeStruct((2, NH, LANES), jnp.float32),
        grid=(2, n_half),
        in_specs=[seg_pair_spec, seg_pair_spec],
        out_specs=seg_out_spec,
        compiler_params=seg_params,
    )(e_2d.reshape(1, P_pad), idx_row)

    return jnp.sum(e_parts, axis=0).reshape(N_pad)[:N]
